# baseline (speedup 1.0000x reference)
import functools

import jax
import jax.numpy as jnp
import numpy as np
from jax import lax
from jax.experimental import pallas as pl
from jax.experimental.pallas import tpu as pltpu
from jax.experimental.pallas import tpu_sc as plsc

_N = 10000
_E = 640000
_DH = 64
_NC = 2
_NS = 16
_K = 128
_CHP = 157
_NPH = 2
_CH = _NPH * _CHP
_NP = 10112
_RZ = _NP // _NS


def _sc_body(tables, src_p, dst_p, zeros, out, src_v, dst_v, rows_a, rows_b,
             msg, sem_ga, sem_gb, sem_sa, sem_sb):
    c = lax.axis_index("c")
    s = lax.axis_index("s")
    tbl = tables.at[c]
    pltpu.sync_copy(zeros, msg.at[pl.ds(s * _RZ, _RZ)])
    plsc.subcore_barrier()

    def pair(j, carry):
        j0 = 2 * j
        ga = pltpu.async_copy(tbl.at[src_v.at[j0]], rows_a, sem_ga)
        gb = pltpu.async_copy(tbl.at[src_v.at[j0 + 1]], rows_b, sem_gb)
        ga.wait()
        sa = pltpu.async_copy(rows_a, msg.at[dst_v.at[j0]], sem_sa, add=True)
        gb.wait()
        sb = pltpu.async_copy(rows_b, msg.at[dst_v.at[j0 + 1]], sem_sb, add=True)
        sa.wait()
        sb.wait()
        return carry

    def phase(p, carry):
        base = s * _CH + p * _CHP
        pltpu.sync_copy(src_p.at[pl.ds(base, _CHP)], src_v)
        pltpu.sync_copy(dst_p.at[pl.ds(base, _CHP)], dst_v)
        lax.fori_loop(0, _CHP // 2, pair, 0)
        gt = pltpu.async_copy(tbl.at[src_v.at[_CHP - 1]], rows_a, sem_ga)
        gt.wait()
        st = pltpu.async_copy(rows_a, msg.at[dst_v.at[_CHP - 1]], sem_sa,
                              add=True)
        st.wait()
        return carry

    lax.fori_loop(0, _NPH, phase, 0)
    plsc.subcore_barrier()
    pltpu.sync_copy(msg.at[pl.ds(s * _RZ, _RZ)], out.at[c, pl.ds(s * _RZ, _RZ)])


def _sc_segment_sum(tables, src_p, dst_p, zeros):
    mesh = plsc.VectorSubcoreMesh(core_axis_name="c", subcore_axis_name="s",
                                  num_cores=_NC, num_subcores=_NS)
    fn = pl.kernel(
        _sc_body,
        out_type=jax.ShapeDtypeStruct((_NC, _NP, _DH), jnp.float32),
        mesh=mesh,
        scratch_types=[
            pltpu.VMEM((_CHP, _K), jnp.int32),
            pltpu.VMEM((_CHP, _K), jnp.int32),
            pltpu.VMEM((_K, _DH), jnp.float32),
            pltpu.VMEM((_K, _DH), jnp.float32),
            pltpu.VMEM_SHARED((_NP, _DH), jnp.float32),
            pltpu.SemaphoreType.DMA,
            pltpu.SemaphoreType.DMA,
            pltpu.SemaphoreType.DMA,
            pltpu.SemaphoreType.DMA,
        ],
        compiler_params=pltpu.CompilerParams(use_tc_tiling_on_sc=False),
    )
    return fn(tables, src_p, dst_p, zeros)


_B = 2000


def _tc_pre_body(xs_ref, xv_ref, fr_ref, w1_ref, o1x_ref, fb_ref):
    xs = xs_ref[...]
    xv = xv_ref[...]
    fr = fr_ref[...]
    w1 = w1_ref[...]
    fb = [jnp.broadcast_to(fr[:, k:k + 1], (fr.shape[0], 16))
          for k in range(9)]
    fb_ref[...] = jnp.concatenate(fb, axis=1)

    def f(i, j):
        return fb[3 * i + j]

    vx = [f(i, 0) * xv[:, 0:16] + f(i, 1) * xv[:, 16:32]
          + f(i, 2) * xv[:, 32:48] for i in range(3)]
    o1x = jnp.dot(xs, w1[64:128], preferred_element_type=jnp.float32)
    for i in range(3):
        o1x = o1x + jnp.dot(vx[i], w1[144 + 32 * i:160 + 32 * i],
                            preferred_element_type=jnp.float32)
    o1x_ref[...] = o1x


def _tc_body(parts_ref, o1x_ref, fb_ref, w1_ref, w2_ref, us_ref, uv_ref):
    ms = parts_ref[0]
    mv = parts_ref[1][:, :48]
    fbv = fb_ref[...]
    w1 = w1_ref[...]
    w2 = w2_ref[...]

    def f(i, j):
        k = 3 * i + j
        return fbv[:, 16 * k:16 * k + 16]

    def dot(a, b):
        return jnp.dot(a, b, preferred_element_type=jnp.float32)

    vm = [f(i, 0) * mv[:, 0:16] + f(i, 1) * mv[:, 16:32]
          + f(i, 2) * mv[:, 32:48] for i in range(3)]
    o1 = o1x_ref[...] + dot(ms, w1[0:64])
    for i in range(3):
        o1 = o1 + dot(vm[i], w1[128 + 32 * i:144 + 32 * i])
    s1 = o1[:, :64]
    m_s = s1 * jax.nn.sigmoid(s1)
    vc = [o1[:, 64 + 16 * i:80 + 16 * i] for i in range(3)]
    mv2 = [f(0, j) * vc[0] + f(1, j) * vc[1] + f(2, j) * vc[2]
           for j in range(3)]
    vloc2 = [f(i, 0) * mv2[0] + f(i, 1) * mv2[1] + f(i, 2) * mv2[2]
             for i in range(3)]
    o2 = dot(m_s, w2[0:64])
    for i in range(3):
        o2 = o2 + dot(vloc2[i], w2[64 + 16 * i:80 + 16 * i])
    s2 = o2[:, :64]
    us_ref[...] = s2 * jax.nn.sigmoid(s2)
    uvc = [o2[:, 64 + 16 * i:80 + 16 * i] for i in range(3)]
    uv_ref[...] = jnp.concatenate(
        [f(0, j) * uvc[0] + f(1, j) * uvc[1] + f(2, j) * uvc[2]
         for j in range(3)], axis=1)


def _tc_pre(x_s, xv_pl, fr9, w1):
    return pl.pallas_call(
        _tc_pre_body,
        grid=(_N // _B,),
        in_specs=[
            pl.BlockSpec((_B, 64), lambda i: (i, 0)),
            pl.BlockSpec((_B, 48), lambda i: (i, 0)),
            pl.BlockSpec((_B, 9), lambda i: (i, 0)),
            pl.BlockSpec((224, 112), lambda i: (0, 0)),
        ],
        out_specs=[
            pl.BlockSpec((_B, 112), lambda i: (i, 0)),
            pl.BlockSpec((_B, 144), lambda i: (i, 0)),
        ],
        out_shape=[
            jax.ShapeDtypeStruct((_N, 112), jnp.float32),
            jax.ShapeDtypeStruct((_N, 144), jnp.float32),
        ],
    )(x_s, xv_pl, fr9, w1)


def _tc_tpp(parts, o1x, fb, w1, w2):
    return pl.pallas_call(
        _tc_body,
        grid=(_N // _B,),
        in_specs=[
            pl.BlockSpec((_NC, _B, _DH), lambda i: (0, i, 0)),
            pl.BlockSpec((_B, 112), lambda i: (i, 0)),
            pl.BlockSpec((_B, 144), lambda i: (i, 0)),
            pl.BlockSpec((224, 112), lambda i: (0, 0)),
            pl.BlockSpec((112, 112), lambda i: (0, 0)),
        ],
        out_specs=[
            pl.BlockSpec((_B, 64), lambda i: (i, 0)),
            pl.BlockSpec((_B, 48), lambda i: (i, 0)),
        ],
        out_shape=[
            jax.ShapeDtypeStruct((_N, 64), jnp.float32),
            jax.ShapeDtypeStruct((_N, 48), jnp.float32),
        ],
    )(parts, o1x, fb, w1, w2)


_CPERM_V = np.arange(48).reshape(16, 3).T.reshape(-1)
_RPERM1 = np.concatenate([np.arange(128),
                          128 + np.arange(96).reshape(32, 3).T.reshape(-1)])
_RPERM2 = np.concatenate([np.arange(64),
                          64 + np.arange(48).reshape(16, 3).T.reshape(-1)])


def kernel(x_s, x_v, edge_index, frames, W_intra_s, W_intra_v, W_inter_s, W_inter_v):
    xv_pl = x_v.transpose(0, 2, 1).reshape(_N, 48)
    tables = jnp.stack(
        [x_s, jnp.concatenate([xv_pl, jnp.zeros((_N, 16), jnp.float32)],
                              axis=1)])
    pad = _NS * _CH * _K - _E
    src_p = jnp.concatenate(
        [edge_index[1].astype(jnp.int32),
         jnp.zeros((pad,), jnp.int32)]).reshape(_NS * _CH, _K)
    dst_p = jnp.concatenate(
        [edge_index[0].astype(jnp.int32),
         jnp.full((pad,), _N, jnp.int32)]).reshape(_NS * _CH, _K)
    zeros = jnp.zeros((_RZ, _DH), jnp.float32)

    parts = _sc_segment_sum(tables, src_p, dst_p, zeros)

    w1 = jnp.concatenate([W_intra_s, W_intra_v[:, _CPERM_V]], axis=1)[_RPERM1]
    w2 = jnp.concatenate([W_inter_s, W_inter_v[:, _CPERM_V]], axis=1)[_RPERM2]

    fr9 = frames.reshape(_N, 9)

    o1x, fb = _tc_pre(x_s, xv_pl, fr9, w1)

    u_s, uv_pl = _tc_tpp(parts, o1x, fb, w1, w2)
    u_v = uv_pl.reshape(_N, 3, 16).transpose(0, 2, 1)
    return u_s, u_v


# speedup vs baseline: 1.0500x; 1.0500x over previous
import functools

import jax
import jax.numpy as jnp
import numpy as np
from jax import lax
from jax.experimental import pallas as pl
from jax.experimental.pallas import tpu as pltpu
from jax.experimental.pallas import tpu_sc as plsc

_N = 10000
_E = 640000
_DH = 64
_NC = 2
_NS = 16
_K = 128
_CHP = 157
_CH = 2 * _CHP
_EPT = _CH * _K
_NP = 10112
_RZ = _NP // _NS


def _sc_body(tables, src_p, dst_p, zeros, out, src_v, dst_v, rows_a, rows_b,
             msg, sem_ga, sem_gb, sem_sa, sem_sb):
    c = lax.axis_index("c")
    s = lax.axis_index("s")
    tbl = tables.at[c]
    pltpu.sync_copy(zeros, msg.at[pl.ds(s * _RZ, _RZ)])
    plsc.subcore_barrier()

    def pair(j, carry):
        j0 = 2 * j
        ga = pltpu.async_copy(tbl.at[src_v.at[j0]], rows_a, sem_ga)
        gb = pltpu.async_copy(tbl.at[src_v.at[j0 + 1]], rows_b, sem_gb)
        ga.wait()
        sa = pltpu.async_copy(rows_a, msg.at[dst_v.at[j0]], sem_sa, add=True)
        gb.wait()
        sb = pltpu.async_copy(rows_b, msg.at[dst_v.at[j0 + 1]], sem_sb, add=True)
        sa.wait()
        sb.wait()
        return carry

    def phase(p, carry):
        pltpu.sync_copy(src_p.at[s, pl.ds(p * _CHP, _CHP)], src_v)
        pltpu.sync_copy(dst_p.at[s, pl.ds(p * _CHP, _CHP)], dst_v)
        lax.fori_loop(0, _CHP // 2, pair, 0)
        gt = pltpu.async_copy(tbl.at[src_v.at[_CHP - 1]], rows_a, sem_ga)
        gt.wait()
        st = pltpu.async_copy(rows_a, msg.at[dst_v.at[_CHP - 1]], sem_sa,
                              add=True)
        st.wait()
        return carry

    lax.fori_loop(0, 2, phase, 0)
    plsc.subcore_barrier()
    pltpu.sync_copy(msg.at[pl.ds(s * _RZ, _RZ)], out.at[c, pl.ds(s * _RZ, _RZ)])


def _sc_segment_sum(tables, src_p, dst_p, zeros):
    mesh = plsc.VectorSubcoreMesh(core_axis_name="c", subcore_axis_name="s",
                                  num_cores=_NC, num_subcores=_NS)
    fn = pl.kernel(
        _sc_body,
        out_type=jax.ShapeDtypeStruct((_NC, _NP, _DH), jnp.float32),
        mesh=mesh,
        scratch_types=[
            pltpu.VMEM((_CHP, _K), jnp.int32),
            pltpu.VMEM((_CHP, _K), jnp.int32),
            pltpu.VMEM((_K, _DH), jnp.float32),
            pltpu.VMEM((_K, _DH), jnp.float32),
            pltpu.VMEM_SHARED((_NP, _DH), jnp.float32),
            pltpu.SemaphoreType.DMA,
            pltpu.SemaphoreType.DMA,
            pltpu.SemaphoreType.DMA,
            pltpu.SemaphoreType.DMA,
        ],
        compiler_params=pltpu.CompilerParams(use_tc_tiling_on_sc=False),
    )
    return fn(tables, src_p, dst_p, zeros)


_B = 1000


def _tc_body(parts_ref, xs_ref, xv_ref, fr_ref, w1_ref, w2_ref, us_ref, uv_ref):
    ms = parts_ref[0]
    mv = parts_ref[1][:, :48]
    xs = xs_ref[...]
    xv = xv_ref[...]
    fr = fr_ref[...]

    def f(i, j):
        return fr[:, 3 * i + j:3 * i + j + 1]

    catv = [jnp.concatenate([mv[:, 16 * j:16 * j + 16],
                             xv[:, 16 * j:16 * j + 16]], axis=1)
            for j in range(3)]
    vloc = [f(i, 0) * catv[0] + f(i, 1) * catv[1] + f(i, 2) * catv[2]
            for i in range(3)]
    feat1 = jnp.concatenate([ms, xs] + vloc, axis=1)
    o1 = jnp.dot(feat1, w1_ref[...], preferred_element_type=jnp.float32)
    s1 = o1[:, :64]
    m_s = s1 * jax.nn.sigmoid(s1)
    vc = [o1[:, 64 + 16 * i:80 + 16 * i] for i in range(3)]
    mv2 = [f(0, j) * vc[0] + f(1, j) * vc[1] + f(2, j) * vc[2]
           for j in range(3)]
    vloc2 = [f(i, 0) * mv2[0] + f(i, 1) * mv2[1] + f(i, 2) * mv2[2]
             for i in range(3)]
    feat2 = jnp.concatenate([m_s] + vloc2, axis=1)
    o2 = jnp.dot(feat2, w2_ref[...], preferred_element_type=jnp.float32)
    s2 = o2[:, :64]
    us_ref[...] = s2 * jax.nn.sigmoid(s2)
    uvc = [o2[:, 64 + 16 * i:80 + 16 * i] for i in range(3)]
    uv_ref[...] = jnp.concatenate(
        [f(0, j) * uvc[0] + f(1, j) * uvc[1] + f(2, j) * uvc[2]
         for j in range(3)], axis=1)


def _tc_tpp(parts, x_s, xv_pl, fr9, w1, w2):
    grid = (_N // _B,)
    return pl.pallas_call(
        _tc_body,
        grid=grid,
        in_specs=[
            pl.BlockSpec((_NC, _B, _DH), lambda i: (0, i, 0)),
            pl.BlockSpec((_B, 64), lambda i: (i, 0)),
            pl.BlockSpec((_B, 48), lambda i: (i, 0)),
            pl.BlockSpec((_B, 9), lambda i: (i, 0)),
            pl.BlockSpec((224, 112), lambda i: (0, 0)),
            pl.BlockSpec((112, 112), lambda i: (0, 0)),
        ],
        out_specs=[
            pl.BlockSpec((_B, 64), lambda i: (i, 0)),
            pl.BlockSpec((_B, 48), lambda i: (i, 0)),
        ],
        out_shape=[
            jax.ShapeDtypeStruct((_N, 64), jnp.float32),
            jax.ShapeDtypeStruct((_N, 48), jnp.float32),
        ],
    )(parts, x_s, xv_pl, fr9, w1, w2)


_CPERM_V = np.arange(48).reshape(16, 3).T.reshape(-1)
_RPERM1 = np.concatenate([np.arange(128),
                          128 + np.arange(96).reshape(32, 3).T.reshape(-1)])
_RPERM2 = np.concatenate([np.arange(64),
                          64 + np.arange(48).reshape(16, 3).T.reshape(-1)])


def kernel(x_s, x_v, edge_index, frames, W_intra_s, W_intra_v, W_inter_s, W_inter_v):
    xv_pl = x_v.transpose(0, 2, 1).reshape(_N, 48)
    tables = jnp.stack(
        [x_s, jnp.concatenate([xv_pl, jnp.zeros((_N, 16), jnp.float32)],
                              axis=1)])
    src = edge_index[1].astype(jnp.int32)
    dst = edge_index[0].astype(jnp.int32)
    pad = _EPT - _E // _NS
    src_p = jnp.concatenate(
        [src.reshape(_NS, _E // _NS),
         jnp.zeros((_NS, pad), jnp.int32)], axis=1).reshape(_NS, _CH, _K)
    dst_p = jnp.concatenate(
        [dst.reshape(_NS, _E // _NS),
         jnp.full((_NS, pad), _N, jnp.int32)], axis=1).reshape(_NS, _CH, _K)
    zeros = jnp.zeros((_RZ, _DH), jnp.float32)

    parts = _sc_segment_sum(tables, src_p, dst_p, zeros)

    w1 = jnp.concatenate([W_intra_s, W_intra_v[:, _CPERM_V]], axis=1)[_RPERM1]
    w2 = jnp.concatenate([W_inter_s, W_inter_v[:, _CPERM_V]], axis=1)[_RPERM2]

    fr9 = frames.reshape(_N, 9)

    u_s, uv_pl = _tc_tpp(parts, x_s, xv_pl, fr9, w1, w2)
    u_v = uv_pl.reshape(_N, 3, 16).transpose(0, 2, 1)
    return u_s, u_v
